# baseline (speedup 1.0000x reference)
import jax
import jax.numpy as jnp
from jax.experimental import pallas as pl
from jax.experimental.pallas import tpu as pltpu

B, C, H, W = 16, 384, 64, 64
D = H * W
R = B * C
S = 16


def _dma_body(idx_ref, x_hbm, out_hbm, sems):
    pending = [None] * S
    for c in range(C):
        slot = c % S
        if pending[slot] is not None:
            pending[slot].wait()
        d = pltpu.make_async_copy(
            x_hbm.at[:, pl.ds(idx_ref[c], 1), :],
            out_hbm.at[:, pl.ds(c, 1), :],
            sems.at[slot],
        )
        d.start()
        pending[slot] = d
    for slot in range(S):
        pending[slot].wait()


@jax.jit
def _shuffle(x3d, perm32):
    f = pl.pallas_call(
        _dma_body,
        grid_spec=pltpu.PrefetchScalarGridSpec(
            num_scalar_prefetch=1,
            grid=(1,),
            in_specs=[pl.BlockSpec(memory_space=pltpu.HBM)],
            out_specs=pl.BlockSpec(memory_space=pltpu.HBM),
            scratch_shapes=[pltpu.SemaphoreType.DMA((S,))],
        ),
        out_shape=jax.ShapeDtypeStruct((B, C, D), jnp.float32),
    )
    return f(perm32, x3d)


def kernel(inputs, permutation):
    x3d = inputs.reshape(B, C, D)
    perm32 = permutation.astype(jnp.int32)
    return _shuffle(x3d, perm32).reshape(B, C, H, W)


# speedup vs baseline: 13.0845x; 13.0845x over previous
import jax
import jax.numpy as jnp
from jax.experimental import pallas as pl
from jax.experimental.pallas import tpu as pltpu

B, C, H, W = 16, 384, 64, 64
D = H * W
RING = 16
LAG = 8


def _dma_body(idx_ref, x_hbm, out_hbm, buf, gsems, ssems):
    pending_in = [None] * RING
    pending_out = [None] * RING

    def start_in(c):
        slot = c % RING
        d = pltpu.make_async_copy(
            x_hbm.at[:, pl.ds(idx_ref[c], 1), :],
            buf.at[slot],
            gsems.at[slot],
        )
        d.start()
        pending_in[slot] = d

    def start_out(c):
        slot = c % RING
        pending_in[slot].wait()
        d = pltpu.make_async_copy(
            buf.at[slot],
            out_hbm.at[:, pl.ds(c, 1), :],
            ssems.at[slot],
        )
        d.start()
        pending_out[slot] = d

    for c in range(C + LAG):
        if c < C:
            slot = c % RING
            if pending_out[slot] is not None:
                pending_out[slot].wait()
                pending_out[slot] = None
            start_in(c)
        if c >= LAG:
            start_out(c - LAG)
    for slot in range(RING):
        if pending_out[slot] is not None:
            pending_out[slot].wait()


@jax.jit
def _shuffle(x3d, perm32):
    f = pl.pallas_call(
        _dma_body,
        grid_spec=pltpu.PrefetchScalarGridSpec(
            num_scalar_prefetch=1,
            grid=(1,),
            in_specs=[pl.BlockSpec(memory_space=pltpu.HBM)],
            out_specs=pl.BlockSpec(memory_space=pltpu.HBM),
            scratch_shapes=[
                pltpu.VMEM((RING, B, 1, D), jnp.float32),
                pltpu.SemaphoreType.DMA((RING,)),
                pltpu.SemaphoreType.DMA((RING,)),
            ],
        ),
        out_shape=jax.ShapeDtypeStruct((B, C, D), jnp.float32),
    )
    return f(perm32, x3d)


def kernel(inputs, permutation):
    x3d = inputs.reshape(B, C, D)
    perm32 = permutation.astype(jnp.int32)
    return _shuffle(x3d, perm32).reshape(B, C, H, W)


# speedup vs baseline: 13.4497x; 1.0279x over previous
import jax
import jax.numpy as jnp
from jax.experimental import pallas as pl
from jax.experimental.pallas import tpu as pltpu

B, C, H, W = 16, 384, 64, 64
D = H * W
RING = 32
LAG = 16


def _dma_body(idx_ref, x_hbm, out_hbm, buf, gsems, ssems):
    pending_in = [None] * RING
    pending_out = [None] * RING

    def start_in(c):
        slot = c % RING
        d = pltpu.make_async_copy(
            x_hbm.at[:, pl.ds(idx_ref[c], 1), :],
            buf.at[slot],
            gsems.at[slot],
        )
        d.start()
        pending_in[slot] = d

    def start_out(c):
        slot = c % RING
        pending_in[slot].wait()
        d = pltpu.make_async_copy(
            buf.at[slot],
            out_hbm.at[:, pl.ds(c, 1), :],
            ssems.at[slot],
        )
        d.start()
        pending_out[slot] = d

    for c in range(C + LAG):
        if c < C:
            slot = c % RING
            if pending_out[slot] is not None:
                pending_out[slot].wait()
                pending_out[slot] = None
            start_in(c)
        if c >= LAG:
            start_out(c - LAG)
    for slot in range(RING):
        if pending_out[slot] is not None:
            pending_out[slot].wait()


@jax.jit
def _shuffle(x3d, perm32):
    f = pl.pallas_call(
        _dma_body,
        grid_spec=pltpu.PrefetchScalarGridSpec(
            num_scalar_prefetch=1,
            grid=(1,),
            in_specs=[pl.BlockSpec(memory_space=pltpu.HBM)],
            out_specs=pl.BlockSpec(memory_space=pltpu.HBM),
            scratch_shapes=[
                pltpu.VMEM((RING, B, 1, D), jnp.float32),
                pltpu.SemaphoreType.DMA((RING,)),
                pltpu.SemaphoreType.DMA((RING,)),
            ],
        ),
        out_shape=jax.ShapeDtypeStruct((B, C, D), jnp.float32),
    )
    return f(perm32, x3d)


def kernel(inputs, permutation):
    x3d = inputs.reshape(B, C, D)
    perm32 = permutation.astype(jnp.int32)
    return _shuffle(x3d, perm32).reshape(B, C, H, W)


# speedup vs baseline: 13.5080x; 1.0043x over previous
import jax
import jax.numpy as jnp
from jax.experimental import pallas as pl
from jax.experimental.pallas import tpu as pltpu

B, C, H, W = 16, 384, 64, 64
D = H * W
RING = 48
LAG = 24


def _dma_body(idx_ref, x_hbm, out_hbm, buf, gsems, ssems):
    pending_in = [None] * RING
    pending_out = [None] * RING

    def start_in(c):
        slot = c % RING
        d = pltpu.make_async_copy(
            x_hbm.at[:, pl.ds(idx_ref[c], 1), :],
            buf.at[slot],
            gsems.at[slot],
        )
        d.start()
        pending_in[slot] = d

    def start_out(c):
        slot = c % RING
        pending_in[slot].wait()
        d = pltpu.make_async_copy(
            buf.at[slot],
            out_hbm.at[:, pl.ds(c, 1), :],
            ssems.at[slot],
        )
        d.start()
        pending_out[slot] = d

    for c in range(C + LAG):
        if c < C:
            slot = c % RING
            if pending_out[slot] is not None:
                pending_out[slot].wait()
                pending_out[slot] = None
            start_in(c)
        if c >= LAG:
            start_out(c - LAG)
    for slot in range(RING):
        if pending_out[slot] is not None:
            pending_out[slot].wait()


@jax.jit
def _shuffle(x3d, perm32):
    f = pl.pallas_call(
        _dma_body,
        grid_spec=pltpu.PrefetchScalarGridSpec(
            num_scalar_prefetch=1,
            grid=(1,),
            in_specs=[pl.BlockSpec(memory_space=pltpu.HBM)],
            out_specs=pl.BlockSpec(memory_space=pltpu.HBM),
            scratch_shapes=[
                pltpu.VMEM((RING, B, 1, D), jnp.float32),
                pltpu.SemaphoreType.DMA((RING,)),
                pltpu.SemaphoreType.DMA((RING,)),
            ],
        ),
        out_shape=jax.ShapeDtypeStruct((B, C, D), jnp.float32),
    )
    return f(perm32, x3d)


def kernel(inputs, permutation):
    x3d = inputs.reshape(B, C, D)
    perm32 = permutation.astype(jnp.int32)
    return _shuffle(x3d, perm32).reshape(B, C, H, W)
